# baseline (speedup 1.0000x reference)
import jax
import jax.numpy as jnp
from jax import lax
from jax.experimental import pallas as pl
from jax.experimental.pallas import tpu as pltpu, tpu_sc as plsc

B = 16384
V = 100000
D = 128
P = 20
N = 50
C = P + N

NC = 2
NS = 16
NW = NC * NS
BPW = B // NW
SR = 64
NSC = BPW // SR
E = SR * C
L = 16


NB = 4
CP = (C + L - 1) // L * L
NG = CP // L
DW = D // 2


def _unpack2(v):
    lo = plsc.bitcast(lax.shift_left(v, 16), jnp.float32)
    hi = plsc.bitcast(jnp.bitwise_and(v, jnp.int32(-65536)), jnp.float32)
    return lo, hi


def _w2v_body(cids_hbm, wid_hbm, wtab_hbm, ctab_hbm, out_hbm,
              cids_v, widx_v, wrows_v, crows_v, dots_v, mat_v, wsem, sems):
    wid = lax.axis_index("s") * NC + lax.axis_index("c")
    base = wid * BPW

    def gather_row(r, b):
        return pltpu.make_async_copy(
            ctab_hbm.at[cids_v.at[r]], crows_v.at[b], sems.at[b]
        )

    def superchunk(sc, _):
        row0 = base + sc * SR
        pltpu.sync_copy(wid_hbm.at[pl.ds(row0, SR)], widx_v)
        pltpu.sync_copy(cids_hbm.at[pl.ds(row0, SR), :], cids_v)
        wcopy = pltpu.make_async_copy(wtab_hbm.at[widx_v], wrows_v, wsem)
        wcopy.start()
        for b in range(NB - 1):
            gather_row(b, b).start()
        wcopy.wait()

        lane = lax.iota(jnp.int32, L)

        def step(r, _):
            b = lax.rem(r, NB)
            gather_row(r, b).wait()

            @pl.when(r + NB - 1 < SR)
            def _():
                nxt = r + NB - 1
                gather_row(nxt, lax.rem(nxt, NB)).start()

            wp = [_unpack2(wrows_v[r, pl.ds(k * L, L)]) for k in range(DW // L)]
            for g in range(NG):
                gvec = jnp.zeros((L,), jnp.float32)
                for i in range(L):
                    j = g * L + i
                    p = []
                    for k in range(DW // L):
                        clo, chi = _unpack2(crows_v[b, j, pl.ds(k * L, L)])
                        p.append(clo * wp[k][0])
                        p.append(chi * wp[k][1])
                    while len(p) > 1:
                        p = [p[z] + p[z + 1] for z in range(0, len(p), 2)]
                    gvec = jnp.where(lane == i, jnp.sum(p[0]), gvec)
                sig = 1.0 / (1.0 + jnp.exp(-gvec))
                dots_v[pl.ds(r * C + g * L, L)] = sig
            return 0

        lax.fori_loop(0, SR, step, 0)
        pltpu.sync_copy(dots_v.at[pl.ds(0, E)], out_hbm.at[pl.ds(row0 * C, E)])
        return 0

    lax.fori_loop(0, NSC, superchunk, 0)


def kernel(word_id, positive_context_ids, negative_context_ids, W_word, W_ctx):
    ctx_ids = jnp.concatenate(
        [positive_context_ids.astype(jnp.int32),
         negative_context_ids.astype(jnp.int32),
         jnp.zeros((B, CP - C), jnp.int32)], axis=1)
    wid32 = word_id.astype(jnp.int32)
    wtab_p = lax.bitcast_convert_type(
        W_word.astype(jnp.bfloat16).reshape(V, DW, 2), jnp.int32)
    ctab_p = lax.bitcast_convert_type(
        W_ctx.astype(jnp.bfloat16).reshape(V, DW, 2), jnp.int32)

    mesh = plsc.VectorSubcoreMesh(core_axis_name="c", subcore_axis_name="s")
    run = pl.kernel(
        _w2v_body,
        out_type=jax.ShapeDtypeStruct((B * C,), jnp.float32),
        mesh=mesh,
        compiler_params=pltpu.CompilerParams(
            needs_layout_passes=False, use_tc_tiling_on_sc=False),
        scratch_types=[
            pltpu.VMEM((SR, CP), jnp.int32),
            pltpu.VMEM((SR,), jnp.int32),
            pltpu.VMEM((SR, DW), jnp.int32),
            pltpu.VMEM((NB, CP, DW), jnp.int32),
            pltpu.VMEM((E + L,), jnp.float32),
            pltpu.VMEM((L * L,), jnp.float32),
            pltpu.SemaphoreType.DMA,
            pltpu.SemaphoreType.DMA((NB,)),
        ],
    )
    out = run(ctx_ids, wid32, wtab_p, ctab_p).reshape(B, C)
    return out[:, :P], out[:, P:]


# speedup vs baseline: 12.2179x; 12.2179x over previous
import jax
import jax.numpy as jnp
from jax import lax
from jax.experimental import pallas as pl
from jax.experimental.pallas import tpu as pltpu, tpu_sc as plsc

B = 16384
V = 100000
D = 128
P = 20
N = 50
C = P + N

NC = 2
NS = 16
NW = NC * NS
BPW = B // NW
SR = 64
NSC = BPW // SR
E = SR * C
L = 16

NB = 4
CP = (C + L - 1) // L * L
NG = CP // L
KW = SR // 2


def _w2v_body(cids_hbm, wid_hbm, wtab_hbm, ctab_hbm, out_hbm,
              cids_v, widx_v, wrows_v, dots_v, crows_v,
              wsem, idsem, osems, sems):
    wid = lax.axis_index("s") * NC + lax.axis_index("c")
    base = wid * BPW

    def stage_ids(s, p):
        row0 = base + s * SR
        a = pltpu.make_async_copy(
            wid_hbm.at[pl.ds(row0, SR)], widx_v.at[p], idsem)
        b = pltpu.make_async_copy(
            cids_hbm.at[pl.ds(row0, SR), :], cids_v.at[p], idsem)
        return a, b

    def word_gather(p):
        return pltpu.make_async_copy(
            wtab_hbm.at[widx_v.at[p]], wrows_v.at[p], wsem)

    def ctx_gather(r, b):
        p = lax.rem(lax.div(r, SR), 2)
        return pltpu.make_async_copy(
            ctab_hbm.at[cids_v.at[p, lax.rem(r, SR)]],
            crows_v.at[b, pl.ds(0, C), :], sems.at[b])

    def out_copy(s, p):
        row0 = base + s * SR
        return pltpu.make_async_copy(
            dots_v.at[p, pl.ds(0, E)], out_hbm.at[pl.ds(row0 * C, E)],
            osems.at[p])

    a0, b0 = stage_ids(0, 0)
    a0.start(); b0.start(); a0.wait(); b0.wait()
    word_gather(0).start()
    for b in range(NB - 1):
        ctx_gather(b, b).start()
    word_gather(0).wait()

    lane = lax.iota(jnp.int32, L)

    def step(r, _):
        s = lax.div(r, SR)
        rr = lax.rem(r, SR)
        p = lax.rem(s, 2)
        b = lax.rem(r, NB)

        @pl.when(jnp.logical_and(rr == 0, s > 0))
        def _():
            word_gather(p).wait()

            @pl.when(s >= 2)
            def _():
                out_copy(s - 2, p).wait()

        @pl.when(jnp.logical_and(rr == 0, s + 1 < NSC))
        def _():
            sa, sb = stage_ids(s + 1, 1 - p)
            sa.start()
            sb.start()

        @pl.when(jnp.logical_and(rr == KW, s + 1 < NSC))
        def _():
            an, bn = stage_ids(s + 1, 1 - p)
            an.wait()
            bn.wait()
            word_gather(1 - p).start()

        ctx_gather(r, b).wait()

        @pl.when(r + NB - 1 < BPW)
        def _():
            nxt = r + NB - 1
            ctx_gather(nxt, lax.rem(nxt, NB)).start()

        wv = [wrows_v[p, rr, pl.ds(k * L, L)] for k in range(D // L)]
        for g in range(NG):
            gvec = jnp.zeros((L,), jnp.float32)
            for i in range(L):
                j = g * L + i
                pr = [crows_v[b, j, pl.ds(k * L, L)] * wv[k]
                      for k in range(D // L)]
                while len(pr) > 1:
                    pr = [pr[z] + pr[z + 1] for z in range(0, len(pr), 2)]
                gvec = jnp.where(lane == i, jnp.sum(pr[0]), gvec)
            sig = 1.0 / (1.0 + jnp.exp(-gvec))
            dots_v[p, pl.ds(rr * C + g * L, L)] = sig

        @pl.when(rr == SR - 1)
        def _():
            out_copy(s, p).start()

        return 0

    lax.fori_loop(0, BPW, step, 0)
    out_copy(NSC - 2, lax.rem(NSC - 2, 2)).wait()
    out_copy(NSC - 1, lax.rem(NSC - 1, 2)).wait()


def kernel(word_id, positive_context_ids, negative_context_ids, W_word, W_ctx):
    ctx_ids = jnp.concatenate(
        [positive_context_ids, negative_context_ids], axis=1
    ).astype(jnp.int32)
    wid32 = word_id.astype(jnp.int32)

    mesh = plsc.VectorSubcoreMesh(core_axis_name="c", subcore_axis_name="s")
    run = pl.kernel(
        _w2v_body,
        out_type=jax.ShapeDtypeStruct((B * C,), jnp.float32),
        mesh=mesh,
        compiler_params=pltpu.CompilerParams(needs_layout_passes=False),
        scratch_types=[
            pltpu.VMEM((2, SR, C), jnp.int32),
            pltpu.VMEM((2, SR), jnp.int32),
            pltpu.VMEM((2, SR, D), jnp.float32),
            pltpu.VMEM((2, E + L), jnp.float32),
            pltpu.VMEM((NB, CP, D), jnp.float32),
            pltpu.SemaphoreType.DMA,
            pltpu.SemaphoreType.DMA,
            pltpu.SemaphoreType.DMA((2,)),
            pltpu.SemaphoreType.DMA((NB,)),
        ],
    )
    out = run(ctx_ids, wid32, W_word, W_ctx).reshape(B, C)
    return out[:, :P], out[:, P:]
